# baseline (speedup 1.0000x reference)
import jax
import jax.numpy as jnp
from jax.experimental import pallas as pl
from jax.experimental.pallas import tpu as pltpu

_N = 10000
_IN_FT = 256
_OUT_FT = 256
_BM = 200


def _gcn_kernel(seq_ref, wt_ref, bfc_ref, adja_ref, adjb_ref, bias_ref,
                ap_ref, out_ref, sf_ref):
    @pl.when(pl.program_id(0) == 0)
    def _():
        sf_ref[...] = (
            jnp.dot(seq_ref[...], wt_ref[...],
                    preferred_element_type=jnp.float32)
            + bfc_ref[...]
        )

    sf = sf_ref[...]
    bias_row = bias_ref[...]
    ap = ap_ref[0, 0]

    acc_a = jnp.dot(adja_ref[...], sf, preferred_element_type=jnp.float32)
    acc_a = acc_a + bias_row
    out_ref[0:_BM, :] = jnp.where(acc_a >= 0.0, acc_a, ap * acc_a)

    acc_b = jnp.dot(adjb_ref[...], sf, preferred_element_type=jnp.float32)
    acc_b = acc_b + bias_row
    out_ref[_BM:2 * _BM, :] = jnp.where(acc_b >= 0.0, acc_b, ap * acc_b)


def kernel(seq, adj, W_fc, b_fc, bias, a_prelu):
    wt = W_fc.T
    bfc2 = b_fc.reshape(1, _OUT_FT)
    bias2 = bias.reshape(1, _OUT_FT)
    ap2 = a_prelu.reshape(1, 1)

    return pl.pallas_call(
        _gcn_kernel,
        grid=(_N // (2 * _BM),),
        in_specs=[
            pl.BlockSpec((_N, _IN_FT), lambda i: (0, 0)),
            pl.BlockSpec((_IN_FT, _OUT_FT), lambda i: (0, 0)),
            pl.BlockSpec((1, _OUT_FT), lambda i: (0, 0)),
            pl.BlockSpec((_BM, _N), lambda i: (2 * i, 0)),
            pl.BlockSpec((_BM, _N), lambda i: (2 * i + 1, 0)),
            pl.BlockSpec((1, _OUT_FT), lambda i: (0, 0)),
            pl.BlockSpec((1, 1), lambda i: (0, 0)),
        ],
        out_specs=pl.BlockSpec((2 * _BM, _OUT_FT), lambda i: (i, 0)),
        out_shape=jax.ShapeDtypeStruct((_N, _OUT_FT), jnp.float32),
        scratch_shapes=[pltpu.VMEM((_N, _OUT_FT), jnp.float32)],
        compiler_params=pltpu.CompilerParams(
            dimension_semantics=("arbitrary",),
        ),
    )(seq, wt, bfc2, adj, adj, bias2, ap2)


# speedup vs baseline: 1.0178x; 1.0178x over previous
import jax
import jax.numpy as jnp
from jax.experimental import pallas as pl
from jax.experimental.pallas import tpu as pltpu

_N = 10000
_IN_FT = 256
_OUT_FT = 256
_BM = 400


def _gcn_kernel(seq_ref, wt_ref, bfc_ref, adj_ref, bias_ref, ap_ref,
                out_ref, sf_ref):
    @pl.when(pl.program_id(0) == 0)
    def _():
        sf_ref[...] = (
            jnp.dot(seq_ref[...], wt_ref[...],
                    preferred_element_type=jnp.float32)
            + bfc_ref[...]
        )

    acc = jnp.dot(adj_ref[...], sf_ref[...],
                  preferred_element_type=jnp.float32)
    acc = acc + bias_ref[...]
    out_ref[...] = jnp.where(acc >= 0.0, acc, ap_ref[0, 0] * acc)


def kernel(seq, adj, W_fc, b_fc, bias, a_prelu):
    wt = W_fc.T
    bfc2 = b_fc.reshape(1, _OUT_FT)
    bias2 = bias.reshape(1, _OUT_FT)
    ap2 = a_prelu.reshape(1, 1)

    return pl.pallas_call(
        _gcn_kernel,
        grid=(_N // _BM,),
        in_specs=[
            pl.BlockSpec((_N, _IN_FT), lambda i: (0, 0)),
            pl.BlockSpec((_IN_FT, _OUT_FT), lambda i: (0, 0)),
            pl.BlockSpec((1, _OUT_FT), lambda i: (0, 0)),
            pl.BlockSpec((_BM, _N), lambda i: (i, 0)),
            pl.BlockSpec((1, _OUT_FT), lambda i: (0, 0)),
            pl.BlockSpec((1, 1), lambda i: (0, 0)),
        ],
        out_specs=pl.BlockSpec((_BM, _OUT_FT), lambda i: (i, 0)),
        out_shape=jax.ShapeDtypeStruct((_N, _OUT_FT), jnp.float32),
        scratch_shapes=[pltpu.VMEM((_N, _OUT_FT), jnp.float32)],
        compiler_params=pltpu.CompilerParams(
            dimension_semantics=("arbitrary",),
        ),
    )(seq, wt, bfc2, adj, bias2, ap2)


# speedup vs baseline: 1.0332x; 1.0152x over previous
import jax
import jax.numpy as jnp
from jax.experimental import pallas as pl
from jax.experimental.pallas import tpu as pltpu

_N = 10000
_IN_FT = 256
_OUT_FT = 256
_BM = 400


def _gcn_kernel(seq_ref, w_ref, bfc_ref, adj_ref, bias_ref, ap_ref,
                out_ref, sf_ref):
    @pl.when(pl.program_id(0) == 0)
    def _():
        sf_ref[...] = (
            jax.lax.dot_general(
                seq_ref[...], w_ref[...],
                dimension_numbers=(((1,), (1,)), ((), ())),
                preferred_element_type=jnp.float32)
            + bfc_ref[...]
        )

    acc = jnp.dot(adj_ref[...], sf_ref[...],
                  preferred_element_type=jnp.float32)
    acc = acc + bias_ref[...]
    out_ref[...] = jnp.where(acc >= 0.0, acc, ap_ref[0, 0] * acc)


def kernel(seq, adj, W_fc, b_fc, bias, a_prelu):
    bfc2 = b_fc.reshape(1, _OUT_FT)
    bias2 = bias.reshape(1, _OUT_FT)
    ap2 = a_prelu.reshape(1, 1)

    return pl.pallas_call(
        _gcn_kernel,
        grid=(_N // _BM,),
        in_specs=[
            pl.BlockSpec((_N, _IN_FT), lambda i: (0, 0)),
            pl.BlockSpec((_IN_FT, _OUT_FT), lambda i: (0, 0)),
            pl.BlockSpec((1, _OUT_FT), lambda i: (0, 0)),
            pl.BlockSpec((_BM, _N), lambda i: (i, 0)),
            pl.BlockSpec((1, _OUT_FT), lambda i: (0, 0)),
            pl.BlockSpec((1, 1), lambda i: (0, 0)),
        ],
        out_specs=pl.BlockSpec((_BM, _OUT_FT), lambda i: (i, 0)),
        out_shape=jax.ShapeDtypeStruct((_N, _OUT_FT), jnp.float32),
        scratch_shapes=[pltpu.VMEM((_N, _OUT_FT), jnp.float32)],
        compiler_params=pltpu.CompilerParams(
            dimension_semantics=("arbitrary",),
        ),
    )(seq, W_fc, bfc2, adj, bias2, ap2)
